# baseline (speedup 1.0000x reference)
import functools

import jax
import jax.numpy as jnp
from jax import lax
from jax.experimental import pallas as pl
from jax.experimental.pallas import tpu as pltpu
from jax.experimental.pallas import tpu_sc as plsc

N = 100000
NBX = 256
NBY = 256
BSX = 1.0 / NBX
BSY = 1.0 / NBY
STRETCH = 1.4142135
MINSX = BSX * STRETCH
MINSY = BSY * STRETCH
INV_CAP = 1.0 / 100.0

NW = 32
PER_W = 3136
NBATCH = PER_W // 16
LAST_START = N - PER_W
LAST_SKIP = (NW - 1) * PER_W - LAST_START
LAST_B0 = LAST_SKIP // 16

ROWS = 264
SROW = 264
HROWS = 384
ACCW = HROWS * SROW
ZEROW = ROWS * SROW


def _build_sc_scatter():
    mesh = plsc.VectorSubcoreMesh(core_axis_name="c", subcore_axis_name="s")

    @functools.partial(
        pl.kernel,
        mesh=mesh,
        compiler_params=pltpu.CompilerParams(needs_layout_passes=False),
        out_type=jax.ShapeDtypeStruct((NW, ACCW), jnp.float32),
        scratch_types=[
            pltpu.VMEM((5 * PER_W,), jnp.float32),
            pltpu.VMEM((ACCW,), jnp.float32),
            pltpu.VMEM((272,), jnp.float32),
            pltpu.VMEM((272,), jnp.int32),
            pltpu.VMEM((272,), jnp.float32),
            pltpu.VMEM((272,), jnp.int32),
            pltpu.SemaphoreType.DMA,
        ],
    )
    def sc_fn(xf, yf, wf, hf, wtf, outp, inbuf, acc, vbufa, ibufa, vbufb,
              ibufb, sem):
        wid = lax.axis_index("s") * 2 + lax.axis_index("c")
        is_last = wid == NW - 1
        row0 = jnp.where(is_last, LAST_START, wid * PER_W)
        b0 = jnp.where(is_last, LAST_B0, 0)
        cps = [
            pltpu.async_copy(
                src.at[pl.ds(row0, PER_W)],
                inbuf.at[pl.ds(k * PER_W, PER_W)],
                sem,
            )
            for k, src in enumerate((xf, yf, wf, hf, wtf))
        ]

        zero16 = jnp.zeros((16,), jnp.float32)

        def zblock(r, carry):
            for k in range(16):
                acc[pl.ds(r * 256 + k * 16, 16)] = zero16
            return carry

        lax.fori_loop(0, ZEROW // 256, zblock, 0, unroll=False)
        for k in range(ZEROW // 256 * 16, ZEROW // 16):
            acc[pl.ds(k * 16, 16)] = zero16
        for cp in cps:
            cp.wait()

        iota = jnp.arange(16, dtype=jnp.int32)
        tidx = [iota * 17 + q for q in range(16)]

        def side(pos, half):
            u8 = pos * 256.0 - half + 8.0
            v8 = u8 + half + half
            iu = u8.astype(jnp.int32)
            fu = u8 - iu.astype(jnp.float32)
            iv = v8.astype(jnp.int32)
            fv = v8 - iv.astype(jnp.float32)
            j0 = iu - 4
            j1 = iv - 4
            c = (j1 - j0) == 1
            p = [j0, j0 + 1, jnp.where(c, j0 - 1, j1), j1 + 1]
            d = [
                1.0 - fu,
                jnp.where(c, fu - 1.0 + fv, fu),
                jnp.where(c, 0.0, fv - 1.0),
                -fv,
            ]
            return p, d

        def do_batch(t, vbuf, ibuf):
            o = t * 16
            x = inbuf[pl.ds(o, 16)]
            y = inbuf[pl.ds(PER_W + o, 16)]
            w = inbuf[pl.ds(2 * PER_W + o, 16)]
            h = inbuf[pl.ds(3 * PER_W + o, 16)]
            wt = inbuf[pl.ds(4 * PER_W + o, 16)]
            sx = jnp.maximum(w, MINSX)
            sy = jnp.maximum(h, MINSY)
            dens = wt * INV_CAP / (sx * sy)
            px, dx = side(x, sx * 128.0)
            py, dy = side(y, sy * 128.0)
            xm = [p * SROW for p in px]
            dxd = [d * dens for d in dx]
            for q in range(16):
                a, b = q >> 2, q & 3
                plsc.store_scatter(ibuf, [tidx[q]], xm[a] + py[b])
                plsc.store_scatter(vbuf, [tidx[q]], dxd[a] * dy[b])

            @plsc.parallel_loop(0, 16, 1, unroll=16)
            def drain(j):
                off = j * 17
                iv = ibuf[pl.ds(off, 16)]
                vv = vbuf[pl.ds(off, 16)]
                plsc.addupdate_scatter(acc, [iv], vv)

        def pair(tp, carry):
            do_batch(tp * 2, vbufa, ibufa)
            do_batch(tp * 2 + 1, vbufb, ibufb)
            return carry

        lax.fori_loop(b0 // 2, NBATCH // 2, pair, 0, unroll=False)
        pltpu.sync_copy(acc, outp.at[wid])

    return sc_fn


def _tc_sum_body(parts_ref, out_ref):
    out_ref[...] = jnp.sum(parts_ref[...], axis=0)


_tc_sum = pl.pallas_call(
    _tc_sum_body,
    out_shape=jax.ShapeDtypeStruct((ACCW,), jnp.float32),
)


def _tc_reduce_body(s_ref, out_ref):
    s = s_ref[...]
    rmask = lax.broadcasted_iota(jnp.int32, (HROWS, SROW), 0) < ROWS
    s = jnp.where(rmask, s, 0.0)
    c_in = lax.broadcasted_iota(jnp.int32, (NBX, HROWS), 1)
    c_out = lax.broadcasted_iota(jnp.int32, (NBX, HROWS), 0)
    amat = (c_in <= c_out + 4).astype(jnp.float32)
    d_in = lax.broadcasted_iota(jnp.int32, (SROW, NBY), 0)
    d_out = lax.broadcasted_iota(jnp.int32, (SROW, NBY), 1)
    bmat = (d_in <= d_out + 4).astype(jnp.float32)
    t = jax.lax.dot(s, bmat, precision=jax.lax.Precision.HIGHEST)
    out_ref[...] = jax.lax.dot(amat, t, precision=jax.lax.Precision.HIGHEST)


_tc_reduce = pl.pallas_call(
    _tc_reduce_body,
    out_shape=jax.ShapeDtypeStruct((NBX, NBY), jnp.float32),
)


def kernel(inst_sizes, inst_pos, inst_pin_weights):
    x = inst_pos[:, 0]
    y = inst_pos[:, 1]
    w = inst_sizes[:, 0]
    h = inst_sizes[:, 1]
    parts = _build_sc_scatter()(x, y, w, h, inst_pin_weights)
    s = _tc_sum(parts).reshape(HROWS, SROW)
    return _tc_reduce(s)
